# baseline (speedup 1.0000x reference)
import jax
import jax.numpy as jnp
from jax.experimental import pallas as pl
from jax.experimental.pallas import tpu as pltpu

_B = 512
_L = 100
_E = 128
_C = 10.0
_G = 64


def _decode_step(
    hnp_hbm,
    tg_hbm,
    hn0,
    g,
    costs,
    ox0, oy0, ox2, oy2,
    ncell,
    hbar,
    Wq, bq, WkT, bkT, Wvw, bvw, iw,
    lp_o,
    rew_o,
    act_o,
    hnp_v,
    kt_s,
    tg_tmp,
    q_s,
    inith_s,
    h_s,
    mask_s,
    s_s,
    prev_s,
    sem_hn,
    sem_tg,
):
    i = pl.program_id(0)
    lane = jax.lax.broadcasted_iota(jnp.int32, (_B, _L), 1)

    @pl.when(i == 0)
    def _init():
        cp = pltpu.make_async_copy(hnp_hbm, hnp_v, sem_hn)
        cp.start()
        pltpu.make_async_copy(tg_hbm.at[0], tg_tmp.at[0], sem_tg.at[0]).start()
        for gi_ in range(_G):
            if gi_ + 1 < _G:
                pltpu.make_async_copy(tg_hbm.at[gi_ + 1],
                                      tg_tmp.at[(gi_ + 1) % 2],
                                      sem_tg.at[(gi_ + 1) % 2]).start()
            pltpu.make_async_copy(tg_hbm.at[gi_], tg_tmp.at[gi_ % 2],
                                  sem_tg.at[gi_ % 2]).wait()
            ktg = jnp.dot(WkT[...], tg_tmp[gi_ % 2]) + bkT[...]
            kt_s[gi_] = ktg.astype(jnp.bfloat16)
        cp.wait()
        q_s[...] = hbar[...] + (jnp.dot(iw[...], Wvw[...]) + bvw[...])
        mask0 = jnp.where(hn0[...] == 0.0, 1.0, 0.0).astype(jnp.float32)
        mask_s[...] = jnp.where(lane < 4, 1.0, mask0)
        prev_s[...] = jnp.zeros((_B, 1), jnp.int32)
        inith_s[...] = jnp.zeros((_B, _E), jnp.float32)
        lp_o[...] = jnp.zeros((_B, 1), jnp.float32)
        rew_o[...] = jnp.zeros((_B, 1), jnp.float32)

    q = jnp.dot(q_s[...], Wq[...]) + bq[...]
    qb = q.astype(jnp.bfloat16)
    sub8 = jax.lax.broadcasted_iota(jnp.int32, (8, 1024), 0)
    lane8 = jax.lax.broadcasted_iota(jnp.int32, (8, 1024), 1)
    keep = (lane8 // 128) == sub8
    for gi_ in range(_G):
        sg = jnp.dot(qb[8 * gi_:8 * (gi_ + 1), :], kt_s[gi_],
                     preferred_element_type=jnp.float32)
        m = jnp.where(keep, sg, 0.0)
        d = m[:, 0:128]
        for j in range(1, 8):
            d = d + m[:, 128 * j:128 * (j + 1)]
        s_s[8 * gi_:8 * (gi_ + 1), :] = d[:, :_L]

    u = _C * jnp.tanh(s_s[...] / jnp.sqrt(jnp.float32(128.0)))
    u = jnp.where(mask_s[...] == 1.0, jnp.float32(-1e8), u)
    mx = jnp.max(u, axis=1, keepdims=True)
    ex = jnp.exp(u - mx)
    v = jnp.concatenate([ex, jnp.zeros((_B, 128 - _L), jnp.float32)], axis=1)
    acc = v[:, 0:8]
    for t in range(1, 16):
        acc = acc + v[:, 8 * t:8 * (t + 1)]
    acc = acc[:, 0:4] + acc[:, 4:8]
    acc = acc[:, 0:2] + acc[:, 2:4]
    den = acc[:, 0:1] + acc[:, 1:2]
    p = ex / den
    lg = jnp.log(p + 1e-12)

    val = lg + g[0]
    mv = jnp.max(val, axis=1, keepdims=True)
    idx_samp = jnp.min(jnp.where(val == mv, lane, 10000), axis=1,
                       keepdims=True)
    c4 = jnp.where(lane < 4, costs[...], jnp.float32(3e38))
    mn = jnp.min(c4, axis=1, keepdims=True)
    idx0 = jnp.min(jnp.where(c4 == mn, lane, 10000), axis=1, keepdims=True)
    idx2 = jnp.where(i == 0, idx0, idx_samp).astype(jnp.int32)

    sel = lane == idx2
    done = ncell[...] <= i
    lp = jnp.sum(jnp.where(sel, lg, 0.0), axis=1, keepdims=True)
    lp = jnp.where(done, 0.0, lp)
    lp_o[...] = lp_o[...] + lp

    mask_s[...] = jnp.where((lane // 4) == (idx2 // 4), 1.0, mask_s[...])

    selp = lane == prev_s[...]
    e0 = jnp.sum(jnp.where(sel, ox0[...], 0.0), axis=1, keepdims=True)
    e1 = jnp.sum(jnp.where(sel, oy0[...], 0.0), axis=1, keepdims=True)
    s2 = jnp.sum(jnp.where(selp, ox2[...], 0.0), axis=1, keepdims=True)
    s3 = jnp.sum(jnp.where(selp, oy2[...], 0.0), axis=1, keepdims=True)
    dx = e0 - s2
    dy = e1 - s3
    ext = jnp.sqrt(dx * dx + dy * dy)
    ci = jnp.sum(jnp.where(sel, costs[...], 0.0), axis=1, keepdims=True)
    cs = jnp.sum(jnp.where(selp, costs[...], 0.0), axis=1, keepdims=True)
    reward = (ext + (cs + ci)) / 70.0
    reward = jnp.where(done, 0.0, reward)
    reward = jnp.where(i == 0, 0.0, reward)
    rew_o[...] = rew_o[...] + reward

    lmod = lane8 - (lane8 // 128) * 128
    for gi_ in range(_G):
        idxg = idx2[8 * gi_:8 * (gi_ + 1), :]
        ohg = jnp.where(keep & (lmod == idxg), 1.0, 0.0).astype(jnp.bfloat16)
        h_s[8 * gi_:8 * (gi_ + 1), :] = jnp.dot(
            ohg, hnp_v[gi_], preferred_element_type=jnp.float32)

    @pl.when(i == 0)
    def _set_init_h():
        inith_s[...] = h_s[...]

    cat = jnp.concatenate([inith_s[...], h_s[...]], axis=1)
    hrest = jnp.dot(cat, Wvw[...]) + bvw[...]
    q_s[...] = hbar[...] + hrest
    prev_s[...] = idx2
    act_o[...] = idx2.reshape(1, _B, 1)


def kernel(high_node, original_node, map, num_cell, costs, init_w,
           W_hc, b_hc, W_vw, b_vw, Wq, bq, Wk, bk):
    f32 = jnp.float32
    hn = high_node.astype(f32)
    hn0 = hn[:, :, 0]

    hn_t = jnp.transpose(hn, (0, 2, 1))
    hn_tp = jnp.pad(hn_t, ((0, 0), (0, 0), (0, 28)))
    hn_tg = jnp.transpose(hn_tp.reshape(_G, 8, _E, 128),
                          (0, 2, 1, 3)).reshape(_G, _E, 1024)

    hnp = jnp.pad(hn, ((0, 0), (0, 28), (0, 0))).astype(jnp.bfloat16)
    hnp = hnp.reshape(_G, 1024, _E)

    hbar = jnp.mean(hn, axis=1) @ W_hc + b_hc

    skey = jax.random.key(123)
    keys = jax.vmap(lambda t: jax.random.fold_in(skey, t))(jnp.arange(_L))
    g = jax.vmap(lambda k: jax.random.gumbel(k, (_B, _L), f32))(keys)

    costs_f = costs.astype(f32)
    on = original_node.astype(f32)
    ox0 = on[:, :, 0]
    oy0 = on[:, :, 1]
    ox2 = on[:, :, 2]
    oy2 = on[:, :, 3]
    iw2 = init_w.reshape(1, 2 * _E).astype(f32)
    bq2 = bq.reshape(1, _E).astype(f32)
    bkT = bk.reshape(_E, 1).astype(f32)
    bvw2 = b_vw.reshape(1, _E).astype(f32)
    WkT = Wk.T

    const2 = pl.BlockSpec((_B, _L), lambda i: (0, 0))
    full = lambda shape: pl.BlockSpec(shape, lambda i: tuple(0 for _ in shape))

    lp, rew, act = pl.pallas_call(
        _decode_step,
        grid=(_L,),
        in_specs=[
            pl.BlockSpec(memory_space=pl.ANY),
            pl.BlockSpec(memory_space=pl.ANY),
            const2,
            pl.BlockSpec((1, _B, _L), lambda i: (i, 0, 0)),
            const2, const2, const2, const2, const2,
            full((_B, 1)),
            full((_B, _E)),
            full((_E, _E)), full((1, _E)),
            full((_E, _E)), full((_E, 1)),
            full((2 * _E, _E)), full((1, _E)),
            full((1, 2 * _E)),
        ],
        out_specs=[
            full((_B, 1)),
            full((_B, 1)),
            pl.BlockSpec((1, _B, 1), lambda i: (i, 0, 0)),
        ],
        out_shape=[
            jax.ShapeDtypeStruct((_B, 1), f32),
            jax.ShapeDtypeStruct((_B, 1), f32),
            jax.ShapeDtypeStruct((_L, _B, 1), jnp.int32),
        ],
        scratch_shapes=[
            pltpu.VMEM((_G, 1024, _E), jnp.bfloat16),
            pltpu.VMEM((_G, _E, 1024), jnp.bfloat16),
            pltpu.VMEM((2, _E, 1024), f32),
            pltpu.VMEM((_B, _E), f32),
            pltpu.VMEM((_B, _E), f32),
            pltpu.VMEM((_B, _E), f32),
            pltpu.VMEM((_B, _L), f32),
            pltpu.VMEM((_B, _L), f32),
            pltpu.VMEM((_B, 1), jnp.int32),
            pltpu.SemaphoreType.DMA,
            pltpu.SemaphoreType.DMA((2,)),
        ],
        compiler_params=pltpu.CompilerParams(
            dimension_semantics=("arbitrary",),
            vmem_limit_bytes=100 * 1024 * 1024,
        ),
    )(hnp, hn_tg, hn0, g, costs_f, ox0, oy0, ox2, oy2, num_cell, hbar,
      Wq, bq2, WkT, bkT, W_vw, bvw2, iw2)

    return (lp.reshape(_B), rew.reshape(_B),
            jnp.transpose(act[:, :, 0]).astype(jnp.int32))


# speedup vs baseline: 1.2783x; 1.2783x over previous
import jax
import jax.numpy as jnp
from jax.experimental import pallas as pl
from jax.experimental.pallas import tpu as pltpu

_B = 512
_L = 100
_E = 128
_C = 10.0
_G = 64


def _decode_step(
    hn_hbm,
    tg_hbm,
    hn0,
    g,
    costs,
    ox0, oy0, ox2, oy2,
    ncell,
    hbar,
    Wq, bq, WkT, bkT, Wvw, bvw, iw,
    lp_o,
    rew_o,
    act_o,
    hn_v,
    kt_s,
    tg_tmp,
    q_s,
    inith_s,
    h_s,
    mask_s,
    s_s,
    prev_s,
    sem_hn,
    sem_tg,
):
    i = pl.program_id(0)
    lane = jax.lax.broadcasted_iota(jnp.int32, (_B, _L), 1)

    @pl.when(i == 0)
    def _init():
        cp = pltpu.make_async_copy(hn_hbm, hn_v, sem_hn)
        cp.start()
        pltpu.make_async_copy(tg_hbm.at[0], tg_tmp.at[0], sem_tg.at[0]).start()
        for gi_ in range(_G):
            if gi_ + 1 < _G:
                pltpu.make_async_copy(tg_hbm.at[gi_ + 1],
                                      tg_tmp.at[(gi_ + 1) % 2],
                                      sem_tg.at[(gi_ + 1) % 2]).start()
            pltpu.make_async_copy(tg_hbm.at[gi_], tg_tmp.at[gi_ % 2],
                                  sem_tg.at[gi_ % 2]).wait()
            ktg = jnp.dot(WkT[...], tg_tmp[gi_ % 2]) + bkT[...]
            kt_s[gi_] = ktg.astype(jnp.bfloat16)
        cp.wait()
        q_s[...] = hbar[...] + (jnp.dot(iw[...], Wvw[...]) + bvw[...])
        mask0 = jnp.where(hn0[...] == 0.0, 1.0, 0.0).astype(jnp.float32)
        mask_s[...] = jnp.where(lane < 4, 1.0, mask0)
        prev_s[...] = jnp.zeros((_B, 1), jnp.int32)
        inith_s[...] = jnp.zeros((_B, _E), jnp.float32)
        lp_o[...] = jnp.zeros((_B, 1), jnp.float32)
        rew_o[...] = jnp.zeros((_B, 1), jnp.float32)

    q = jnp.dot(q_s[...], Wq[...]) + bq[...]
    qb = q.astype(jnp.bfloat16)
    sub8 = jax.lax.broadcasted_iota(jnp.int32, (8, 1024), 0)
    lane8 = jax.lax.broadcasted_iota(jnp.int32, (8, 1024), 1)
    keep = (lane8 // 128) == sub8
    for gi_ in range(_G):
        sg = jnp.dot(qb[8 * gi_:8 * (gi_ + 1), :], kt_s[gi_],
                     preferred_element_type=jnp.float32)
        m = jnp.where(keep, sg, 0.0)
        d = m[:, 0:128]
        for j in range(1, 8):
            d = d + m[:, 128 * j:128 * (j + 1)]
        s_s[8 * gi_:8 * (gi_ + 1), :] = d[:, :_L]

    u = _C * jnp.tanh(s_s[...] / jnp.sqrt(jnp.float32(128.0)))
    u = jnp.where(mask_s[...] == 1.0, jnp.float32(-1e8), u)
    mx = jnp.max(u, axis=1, keepdims=True)
    ex = jnp.exp(u - mx)
    v = jnp.concatenate([ex, jnp.zeros((_B, 128 - _L), jnp.float32)], axis=1)
    acc = v[:, 0:8]
    for t in range(1, 16):
        acc = acc + v[:, 8 * t:8 * (t + 1)]
    acc = acc[:, 0:4] + acc[:, 4:8]
    acc = acc[:, 0:2] + acc[:, 2:4]
    den = acc[:, 0:1] + acc[:, 1:2]
    p = ex / den
    lg = jnp.log(p + 1e-12)

    val = lg + g[0]
    mv = jnp.max(val, axis=1, keepdims=True)
    idx_samp = jnp.min(jnp.where(val == mv, lane, 10000), axis=1,
                       keepdims=True)
    c4 = jnp.where(lane < 4, costs[...], jnp.float32(3e38))
    mn = jnp.min(c4, axis=1, keepdims=True)
    idx0 = jnp.min(jnp.where(c4 == mn, lane, 10000), axis=1, keepdims=True)
    idx2 = jnp.where(i == 0, idx0, idx_samp).astype(jnp.int32)

    sel = lane == idx2
    done = ncell[...] <= i
    lp = jnp.sum(jnp.where(sel, lg, 0.0), axis=1, keepdims=True)
    lp = jnp.where(done, 0.0, lp)
    lp_o[...] = lp_o[...] + lp

    mask_s[...] = jnp.where((lane // 4) == (idx2 // 4), 1.0, mask_s[...])

    selp = lane == prev_s[...]
    e0 = jnp.sum(jnp.where(sel, ox0[...], 0.0), axis=1, keepdims=True)
    e1 = jnp.sum(jnp.where(sel, oy0[...], 0.0), axis=1, keepdims=True)
    s2 = jnp.sum(jnp.where(selp, ox2[...], 0.0), axis=1, keepdims=True)
    s3 = jnp.sum(jnp.where(selp, oy2[...], 0.0), axis=1, keepdims=True)
    dx = e0 - s2
    dy = e1 - s3
    ext = jnp.sqrt(dx * dx + dy * dy)
    ci = jnp.sum(jnp.where(sel, costs[...], 0.0), axis=1, keepdims=True)
    cs = jnp.sum(jnp.where(selp, costs[...], 0.0), axis=1, keepdims=True)
    reward = (ext + (cs + ci)) / 70.0
    reward = jnp.where(done, 0.0, reward)
    reward = jnp.where(i == 0, 0.0, reward)
    rew_o[...] = rew_o[...] + reward

    for c in range(16):
        bs = _B // 16
        hc = hn_v[bs * c:bs * (c + 1), :, :]
        li = jax.lax.broadcasted_iota(jnp.int32, (bs, _L, 1), 1)
        idc = idx2[bs * c:bs * (c + 1), :].reshape(bs, 1, 1)
        h_s[bs * c:bs * (c + 1), :] = jnp.sum(
            jnp.where(li == idc, hc, 0.0), axis=1)

    @pl.when(i == 0)
    def _set_init_h():
        inith_s[...] = h_s[...]

    cat = jnp.concatenate([inith_s[...], h_s[...]], axis=1)
    hrest = jnp.dot(cat, Wvw[...]) + bvw[...]
    q_s[...] = hbar[...] + hrest
    prev_s[...] = idx2
    act_o[...] = idx2.reshape(1, _B, 1)


def kernel(high_node, original_node, map, num_cell, costs, init_w,
           W_hc, b_hc, W_vw, b_vw, Wq, bq, Wk, bk):
    f32 = jnp.float32
    hn = high_node.astype(f32)
    hn0 = hn[:, :, 0]

    hn_t = jnp.transpose(hn, (0, 2, 1))
    hn_tp = jnp.pad(hn_t, ((0, 0), (0, 0), (0, 28)))
    hn_tg = jnp.transpose(hn_tp.reshape(_G, 8, _E, 128),
                          (0, 2, 1, 3)).reshape(_G, _E, 1024)

    hbar = jnp.mean(hn, axis=1) @ W_hc + b_hc

    skey = jax.random.key(123)
    keys = jax.vmap(lambda t: jax.random.fold_in(skey, t))(jnp.arange(_L))
    g = jax.vmap(lambda k: jax.random.gumbel(k, (_B, _L), f32))(keys)

    costs_f = costs.astype(f32)
    on = original_node.astype(f32)
    ox0 = on[:, :, 0]
    oy0 = on[:, :, 1]
    ox2 = on[:, :, 2]
    oy2 = on[:, :, 3]
    iw2 = init_w.reshape(1, 2 * _E).astype(f32)
    bq2 = bq.reshape(1, _E).astype(f32)
    bkT = bk.reshape(_E, 1).astype(f32)
    bvw2 = b_vw.reshape(1, _E).astype(f32)
    WkT = Wk.T

    const2 = pl.BlockSpec((_B, _L), lambda i: (0, 0))
    full = lambda shape: pl.BlockSpec(shape, lambda i: tuple(0 for _ in shape))

    lp, rew, act = pl.pallas_call(
        _decode_step,
        grid=(_L,),
        in_specs=[
            pl.BlockSpec(memory_space=pl.ANY),
            pl.BlockSpec(memory_space=pl.ANY),
            const2,
            pl.BlockSpec((1, _B, _L), lambda i: (i, 0, 0)),
            const2, const2, const2, const2, const2,
            full((_B, 1)),
            full((_B, _E)),
            full((_E, _E)), full((1, _E)),
            full((_E, _E)), full((_E, 1)),
            full((2 * _E, _E)), full((1, _E)),
            full((1, 2 * _E)),
        ],
        out_specs=[
            full((_B, 1)),
            full((_B, 1)),
            pl.BlockSpec((1, _B, 1), lambda i: (i, 0, 0)),
        ],
        out_shape=[
            jax.ShapeDtypeStruct((_B, 1), f32),
            jax.ShapeDtypeStruct((_B, 1), f32),
            jax.ShapeDtypeStruct((_L, _B, 1), jnp.int32),
        ],
        scratch_shapes=[
            pltpu.VMEM((_B, _L, _E), f32),
            pltpu.VMEM((_G, _E, 1024), jnp.bfloat16),
            pltpu.VMEM((2, _E, 1024), f32),
            pltpu.VMEM((_B, _E), f32),
            pltpu.VMEM((_B, _E), f32),
            pltpu.VMEM((_B, _E), f32),
            pltpu.VMEM((_B, _L), f32),
            pltpu.VMEM((_B, _L), f32),
            pltpu.VMEM((_B, 1), jnp.int32),
            pltpu.SemaphoreType.DMA,
            pltpu.SemaphoreType.DMA((2,)),
        ],
        compiler_params=pltpu.CompilerParams(
            dimension_semantics=("arbitrary",),
            vmem_limit_bytes=100 * 1024 * 1024,
        ),
    )(hn, hn_tg, hn0, g, costs_f, ox0, oy0, ox2, oy2, num_cell, hbar,
      Wq, bq2, WkT, bkT, W_vw, bvw2, iw2)

    return (lp.reshape(_B), rew.reshape(_B),
            jnp.transpose(act[:, :, 0]).astype(jnp.int32))


# speedup vs baseline: 1.2789x; 1.0004x over previous
import jax
import jax.numpy as jnp
from jax.experimental import pallas as pl
from jax.experimental.pallas import tpu as pltpu

_B = 512
_L = 100
_E = 128
_C = 10.0
_G = 64


def _decode_step(
    hn_hbm,
    tg_hbm,
    hn0,
    g,
    costs,
    ox0, oy0, ox2, oy2,
    ncell,
    hbar,
    Wq, bq, WkT, bkT, Wvw, bvw, iw,
    lp_o,
    rew_o,
    act_o,
    hn_v,
    kt_s,
    tg_tmp,
    q_s,
    inith_s,
    h_s,
    mask_s,
    s_s,
    sg_s,
    prev_s,
    sem_hn,
    sem_tg,
):
    i = pl.program_id(0)
    lane = jax.lax.broadcasted_iota(jnp.int32, (_B, _L), 1)

    @pl.when(i == 0)
    def _init():
        cp = pltpu.make_async_copy(hn_hbm, hn_v, sem_hn)
        cp.start()
        pltpu.make_async_copy(tg_hbm.at[0], tg_tmp.at[0], sem_tg.at[0]).start()
        for gi_ in range(_G):
            if gi_ + 1 < _G:
                pltpu.make_async_copy(tg_hbm.at[gi_ + 1],
                                      tg_tmp.at[(gi_ + 1) % 2],
                                      sem_tg.at[(gi_ + 1) % 2]).start()
            pltpu.make_async_copy(tg_hbm.at[gi_], tg_tmp.at[gi_ % 2],
                                  sem_tg.at[gi_ % 2]).wait()
            ktg = jnp.dot(WkT[...], tg_tmp[gi_ % 2]) + bkT[...]
            kt_s[gi_] = ktg.astype(jnp.bfloat16)
        cp.wait()
        q_s[...] = hbar[...] + (jnp.dot(iw[...], Wvw[...]) + bvw[...])
        mask0 = jnp.where(hn0[...] == 0.0, 1.0, 0.0).astype(jnp.float32)
        mask_s[...] = jnp.where(lane < 4, 1.0, mask0)
        prev_s[...] = jnp.zeros((_B, 1), jnp.int32)
        inith_s[...] = jnp.zeros((_B, _E), jnp.float32)
        lp_o[...] = jnp.zeros((_B, 1), jnp.float32)
        rew_o[...] = jnp.zeros((_B, 1), jnp.float32)

    q = jnp.dot(q_s[...], Wq[...]) + bq[...]
    qb = q.astype(jnp.bfloat16)
    for gi_ in range(_G):
        sg_s[8 * gi_:8 * (gi_ + 1), :] = jnp.dot(
            qb[8 * gi_:8 * (gi_ + 1), :], kt_s[gi_],
            preferred_element_type=jnp.float32)
    subB = jax.lax.broadcasted_iota(jnp.int32, (_B, 1024), 0)
    laneB = jax.lax.broadcasted_iota(jnp.int32, (_B, 1024), 1)
    keepB = (laneB // 128) == (subB - (subB // 8) * 8)
    m = jnp.where(keepB, sg_s[...], 0.0)
    d = m[:, 0:128]
    for j in range(1, 8):
        d = d + m[:, 128 * j:128 * (j + 1)]
    s_s[...] = d[:, :_L]

    u = _C * jnp.tanh(s_s[...] / jnp.sqrt(jnp.float32(128.0)))
    u = jnp.where(mask_s[...] == 1.0, jnp.float32(-1e8), u)
    mx = jnp.max(u, axis=1, keepdims=True)
    ex = jnp.exp(u - mx)
    v = jnp.concatenate([ex, jnp.zeros((_B, 128 - _L), jnp.float32)], axis=1)
    acc = v[:, 0:8]
    for t in range(1, 16):
        acc = acc + v[:, 8 * t:8 * (t + 1)]
    acc = acc[:, 0:4] + acc[:, 4:8]
    acc = acc[:, 0:2] + acc[:, 2:4]
    den = acc[:, 0:1] + acc[:, 1:2]
    p = ex / den
    lg = jnp.log(p + 1e-12)

    val = lg + g[0]
    mv = jnp.max(val, axis=1, keepdims=True)
    idx_samp = jnp.min(jnp.where(val == mv, lane, 10000), axis=1,
                       keepdims=True)
    c4 = jnp.where(lane < 4, costs[...], jnp.float32(3e38))
    mn = jnp.min(c4, axis=1, keepdims=True)
    idx0 = jnp.min(jnp.where(c4 == mn, lane, 10000), axis=1, keepdims=True)
    idx2 = jnp.where(i == 0, idx0, idx_samp).astype(jnp.int32)

    sel = lane == idx2
    done = ncell[...] <= i
    lp = jnp.sum(jnp.where(sel, lg, 0.0), axis=1, keepdims=True)
    lp = jnp.where(done, 0.0, lp)
    lp_o[...] = lp_o[...] + lp

    mask_s[...] = jnp.where((lane // 4) == (idx2 // 4), 1.0, mask_s[...])

    selp = lane == prev_s[...]
    e0 = jnp.sum(jnp.where(sel, ox0[...], 0.0), axis=1, keepdims=True)
    e1 = jnp.sum(jnp.where(sel, oy0[...], 0.0), axis=1, keepdims=True)
    s2 = jnp.sum(jnp.where(selp, ox2[...], 0.0), axis=1, keepdims=True)
    s3 = jnp.sum(jnp.where(selp, oy2[...], 0.0), axis=1, keepdims=True)
    dx = e0 - s2
    dy = e1 - s3
    ext = jnp.sqrt(dx * dx + dy * dy)
    ci = jnp.sum(jnp.where(sel, costs[...], 0.0), axis=1, keepdims=True)
    cs = jnp.sum(jnp.where(selp, costs[...], 0.0), axis=1, keepdims=True)
    reward = (ext + (cs + ci)) / 70.0
    reward = jnp.where(done, 0.0, reward)
    reward = jnp.where(i == 0, 0.0, reward)
    rew_o[...] = rew_o[...] + reward

    for c in range(16):
        bs = _B // 16
        hc = hn_v[bs * c:bs * (c + 1), :, :]
        li = jax.lax.broadcasted_iota(jnp.int32, (bs, _L, 1), 1)
        idc = idx2[bs * c:bs * (c + 1), :].reshape(bs, 1, 1)
        h_s[bs * c:bs * (c + 1), :] = jnp.sum(
            jnp.where(li == idc, hc, 0.0), axis=1)

    @pl.when(i == 0)
    def _set_init_h():
        inith_s[...] = h_s[...]

    cat = jnp.concatenate([inith_s[...], h_s[...]], axis=1)
    hrest = jnp.dot(cat, Wvw[...]) + bvw[...]
    q_s[...] = hbar[...] + hrest
    prev_s[...] = idx2
    act_o[...] = idx2.reshape(1, _B, 1)


def kernel(high_node, original_node, map, num_cell, costs, init_w,
           W_hc, b_hc, W_vw, b_vw, Wq, bq, Wk, bk):
    f32 = jnp.float32
    hn = high_node.astype(f32)
    hn0 = hn[:, :, 0]

    hn_t = jnp.transpose(hn, (0, 2, 1))
    hn_tp = jnp.pad(hn_t, ((0, 0), (0, 0), (0, 28)))
    hn_tg = jnp.transpose(hn_tp.reshape(_G, 8, _E, 128),
                          (0, 2, 1, 3)).reshape(_G, _E, 1024)

    hbar = jnp.mean(hn, axis=1) @ W_hc + b_hc

    skey = jax.random.key(123)
    keys = jax.vmap(lambda t: jax.random.fold_in(skey, t))(jnp.arange(_L))
    g = jax.vmap(lambda k: jax.random.gumbel(k, (_B, _L), f32))(keys)

    costs_f = costs.astype(f32)
    on = original_node.astype(f32)
    ox0 = on[:, :, 0]
    oy0 = on[:, :, 1]
    ox2 = on[:, :, 2]
    oy2 = on[:, :, 3]
    iw2 = init_w.reshape(1, 2 * _E).astype(f32)
    bq2 = bq.reshape(1, _E).astype(f32)
    bkT = bk.reshape(_E, 1).astype(f32)
    bvw2 = b_vw.reshape(1, _E).astype(f32)
    WkT = Wk.T

    const2 = pl.BlockSpec((_B, _L), lambda i: (0, 0))
    full = lambda shape: pl.BlockSpec(shape, lambda i: tuple(0 for _ in shape))

    lp, rew, act = pl.pallas_call(
        _decode_step,
        grid=(_L,),
        in_specs=[
            pl.BlockSpec(memory_space=pl.ANY),
            pl.BlockSpec(memory_space=pl.ANY),
            const2,
            pl.BlockSpec((1, _B, _L), lambda i: (i, 0, 0)),
            const2, const2, const2, const2, const2,
            full((_B, 1)),
            full((_B, _E)),
            full((_E, _E)), full((1, _E)),
            full((_E, _E)), full((_E, 1)),
            full((2 * _E, _E)), full((1, _E)),
            full((1, 2 * _E)),
        ],
        out_specs=[
            full((_B, 1)),
            full((_B, 1)),
            pl.BlockSpec((1, _B, 1), lambda i: (i, 0, 0)),
        ],
        out_shape=[
            jax.ShapeDtypeStruct((_B, 1), f32),
            jax.ShapeDtypeStruct((_B, 1), f32),
            jax.ShapeDtypeStruct((_L, _B, 1), jnp.int32),
        ],
        scratch_shapes=[
            pltpu.VMEM((_B, _L, _E), f32),
            pltpu.VMEM((_G, _E, 1024), jnp.bfloat16),
            pltpu.VMEM((2, _E, 1024), f32),
            pltpu.VMEM((_B, _E), f32),
            pltpu.VMEM((_B, _E), f32),
            pltpu.VMEM((_B, _E), f32),
            pltpu.VMEM((_B, _L), f32),
            pltpu.VMEM((_B, _L), f32),
            pltpu.VMEM((_B, 1024), f32),
            pltpu.VMEM((_B, 1), jnp.int32),
            pltpu.SemaphoreType.DMA,
            pltpu.SemaphoreType.DMA((2,)),
        ],
        compiler_params=pltpu.CompilerParams(
            dimension_semantics=("arbitrary",),
            vmem_limit_bytes=100 * 1024 * 1024,
        ),
    )(hn, hn_tg, hn0, g, costs_f, ox0, oy0, ox2, oy2, num_cell, hbar,
      Wq, bq2, WkT, bkT, W_vw, bvw2, iw2)

    return (lp.reshape(_B), rew.reshape(_B),
            jnp.transpose(act[:, :, 0]).astype(jnp.int32))


# speedup vs baseline: 1.2793x; 1.0004x over previous
import jax
import jax.numpy as jnp
from jax.experimental import pallas as pl
from jax.experimental.pallas import tpu as pltpu

_B = 512
_L = 100
_E = 128
_C = 10.0
_G = 64


def _decode_step(
    hn_hbm,
    tg_hbm,
    hn0,
    g,
    costs,
    ox0, oy0, ox2, oy2,
    ncell,
    hbar,
    Wq, bq, WkT, bkT, Wvw, bvw, iw,
    lp_o,
    rew_o,
    act_o,
    hn_v,
    kt_s,
    tg_tmp,
    q_s,
    inith_s,
    h_s,
    mask_s,
    s_s,
    sg_s,
    prev_s,
    sem_hn,
    sem_tg,
):
    i = pl.program_id(0)
    lane = jax.lax.broadcasted_iota(jnp.int32, (_B, _L), 1)

    @pl.when(i == 0)
    def _init():
        cp = pltpu.make_async_copy(hn_hbm, hn_v, sem_hn)
        cp.start()
        pltpu.make_async_copy(tg_hbm.at[0], tg_tmp.at[0], sem_tg.at[0]).start()
        for gi_ in range(_G):
            if gi_ + 1 < _G:
                pltpu.make_async_copy(tg_hbm.at[gi_ + 1],
                                      tg_tmp.at[(gi_ + 1) % 2],
                                      sem_tg.at[(gi_ + 1) % 2]).start()
            pltpu.make_async_copy(tg_hbm.at[gi_], tg_tmp.at[gi_ % 2],
                                  sem_tg.at[gi_ % 2]).wait()
            ktg = jnp.dot(WkT[...], tg_tmp[gi_ % 2]) + bkT[...]
            kt_s[gi_] = ktg.astype(jnp.bfloat16)
        cp.wait()
        q_s[...] = hbar[...] + (jnp.dot(iw[...], Wvw[...]) + bvw[...])
        mask0 = jnp.where(hn0[...] == 0.0, 1.0, 0.0).astype(jnp.float32)
        mask_s[...] = jnp.where(lane < 4, 1.0, mask0)
        prev_s[...] = jnp.zeros((_B, 1), jnp.int32)
        inith_s[...] = jnp.zeros((_B, _E), jnp.float32)
        lp_o[...] = jnp.zeros((_B, 1), jnp.float32)
        rew_o[...] = jnp.zeros((_B, 1), jnp.float32)

    q = jnp.dot(q_s[...], Wq[...]) + bq[...]
    qb = q.astype(jnp.bfloat16)
    for gi_ in range(_G):
        sg_s[8 * gi_:8 * (gi_ + 1), :] = jnp.dot(
            qb[8 * gi_:8 * (gi_ + 1), :], kt_s[gi_],
            preferred_element_type=jnp.float32)
    subB = jax.lax.broadcasted_iota(jnp.int32, (_B, 1024), 0)
    laneB = jax.lax.broadcasted_iota(jnp.int32, (_B, 1024), 1)
    keepB = (laneB // 128) == (subB - (subB // 8) * 8)
    m = jnp.where(keepB, sg_s[...], 0.0)
    d = m[:, 0:128]
    for j in range(1, 8):
        d = d + m[:, 128 * j:128 * (j + 1)]
    s_s[...] = d[:, :_L]

    u = _C * jnp.tanh(s_s[...] / jnp.sqrt(jnp.float32(128.0)))
    u = jnp.where(mask_s[...] == 1.0, jnp.float32(-1e8), u)
    mx = jnp.max(u, axis=1, keepdims=True)
    ex = jnp.exp(u - mx)
    v = jnp.concatenate([ex, jnp.zeros((_B, 128 - _L), jnp.float32)], axis=1)
    acc = v[:, 0:8]
    for t in range(1, 16):
        acc = acc + v[:, 8 * t:8 * (t + 1)]
    acc = acc[:, 0:4] + acc[:, 4:8]
    acc = acc[:, 0:2] + acc[:, 2:4]
    den = acc[:, 0:1] + acc[:, 1:2]
    p = ex / den
    lg = jnp.log(p + 1e-12)

    val = lg + g[0]
    mv = jnp.max(val, axis=1, keepdims=True)
    idx_samp = jnp.min(jnp.where(val == mv, lane, 10000), axis=1,
                       keepdims=True)
    c4 = jnp.where(lane < 4, costs[...], jnp.float32(3e38))
    mn = jnp.min(c4, axis=1, keepdims=True)
    idx0 = jnp.min(jnp.where(c4 == mn, lane, 10000), axis=1, keepdims=True)
    idx2 = jnp.where(i == 0, idx0, idx_samp).astype(jnp.int32)

    sel = lane == idx2
    done = ncell[...] <= i
    lp = jnp.sum(jnp.where(sel, lg, 0.0), axis=1, keepdims=True)
    lp = jnp.where(done, 0.0, lp)
    lp_o[...] = lp_o[...] + lp

    mask_s[...] = jnp.where((lane // 4) == (idx2 // 4), 1.0, mask_s[...])

    selp = lane == prev_s[...]
    e0 = jnp.sum(jnp.where(sel, ox0[...], 0.0), axis=1, keepdims=True)
    e1 = jnp.sum(jnp.where(sel, oy0[...], 0.0), axis=1, keepdims=True)
    s2 = jnp.sum(jnp.where(selp, ox2[...], 0.0), axis=1, keepdims=True)
    s3 = jnp.sum(jnp.where(selp, oy2[...], 0.0), axis=1, keepdims=True)
    dx = e0 - s2
    dy = e1 - s3
    ext = jnp.sqrt(dx * dx + dy * dy)
    ci = jnp.sum(jnp.where(sel, costs[...], 0.0), axis=1, keepdims=True)
    cs = jnp.sum(jnp.where(selp, costs[...], 0.0), axis=1, keepdims=True)
    reward = (ext + (cs + ci)) / 70.0
    reward = jnp.where(done, 0.0, reward)
    reward = jnp.where(i == 0, 0.0, reward)
    rew_o[...] = rew_o[...] + reward

    for c in range(4):
        bs = _B // 4
        hc = hn_v[bs * c:bs * (c + 1), :, :]
        li = jax.lax.broadcasted_iota(jnp.int32, (bs, _L, 1), 1)
        idc = idx2[bs * c:bs * (c + 1), :].reshape(bs, 1, 1)
        h_s[bs * c:bs * (c + 1), :] = jnp.sum(
            jnp.where(li == idc, hc, 0.0), axis=1)

    @pl.when(i == 0)
    def _set_init_h():
        inith_s[...] = h_s[...]

    cat = jnp.concatenate([inith_s[...], h_s[...]], axis=1)
    hrest = jnp.dot(cat, Wvw[...]) + bvw[...]
    q_s[...] = hbar[...] + hrest
    prev_s[...] = idx2
    act_o[...] = idx2.reshape(1, _B, 1)


def kernel(high_node, original_node, map, num_cell, costs, init_w,
           W_hc, b_hc, W_vw, b_vw, Wq, bq, Wk, bk):
    f32 = jnp.float32
    hn = high_node.astype(f32)
    hn0 = hn[:, :, 0]

    hn_t = jnp.transpose(hn, (0, 2, 1))
    hn_tp = jnp.pad(hn_t, ((0, 0), (0, 0), (0, 28)))
    hn_tg = jnp.transpose(hn_tp.reshape(_G, 8, _E, 128),
                          (0, 2, 1, 3)).reshape(_G, _E, 1024)

    hbar = jnp.mean(hn, axis=1) @ W_hc + b_hc

    skey = jax.random.key(123)
    keys = jax.vmap(lambda t: jax.random.fold_in(skey, t))(jnp.arange(_L))
    g = jax.vmap(lambda k: jax.random.gumbel(k, (_B, _L), f32))(keys)

    costs_f = costs.astype(f32)
    on = original_node.astype(f32)
    ox0 = on[:, :, 0]
    oy0 = on[:, :, 1]
    ox2 = on[:, :, 2]
    oy2 = on[:, :, 3]
    iw2 = init_w.reshape(1, 2 * _E).astype(f32)
    bq2 = bq.reshape(1, _E).astype(f32)
    bkT = bk.reshape(_E, 1).astype(f32)
    bvw2 = b_vw.reshape(1, _E).astype(f32)
    WkT = Wk.T

    const2 = pl.BlockSpec((_B, _L), lambda i: (0, 0))
    full = lambda shape: pl.BlockSpec(shape, lambda i: tuple(0 for _ in shape))

    lp, rew, act = pl.pallas_call(
        _decode_step,
        grid=(_L,),
        in_specs=[
            pl.BlockSpec(memory_space=pl.ANY),
            pl.BlockSpec(memory_space=pl.ANY),
            const2,
            pl.BlockSpec((1, _B, _L), lambda i: (i, 0, 0)),
            const2, const2, const2, const2, const2,
            full((_B, 1)),
            full((_B, _E)),
            full((_E, _E)), full((1, _E)),
            full((_E, _E)), full((_E, 1)),
            full((2 * _E, _E)), full((1, _E)),
            full((1, 2 * _E)),
        ],
        out_specs=[
            full((_B, 1)),
            full((_B, 1)),
            pl.BlockSpec((1, _B, 1), lambda i: (i, 0, 0)),
        ],
        out_shape=[
            jax.ShapeDtypeStruct((_B, 1), f32),
            jax.ShapeDtypeStruct((_B, 1), f32),
            jax.ShapeDtypeStruct((_L, _B, 1), jnp.int32),
        ],
        scratch_shapes=[
            pltpu.VMEM((_B, _L, _E), f32),
            pltpu.VMEM((_G, _E, 1024), jnp.bfloat16),
            pltpu.VMEM((2, _E, 1024), f32),
            pltpu.VMEM((_B, _E), f32),
            pltpu.VMEM((_B, _E), f32),
            pltpu.VMEM((_B, _E), f32),
            pltpu.VMEM((_B, _L), f32),
            pltpu.VMEM((_B, _L), f32),
            pltpu.VMEM((_B, 1024), f32),
            pltpu.VMEM((_B, 1), jnp.int32),
            pltpu.SemaphoreType.DMA,
            pltpu.SemaphoreType.DMA((2,)),
        ],
        compiler_params=pltpu.CompilerParams(
            dimension_semantics=("arbitrary",),
            vmem_limit_bytes=100 * 1024 * 1024,
        ),
    )(hn, hn_tg, hn0, g, costs_f, ox0, oy0, ox2, oy2, num_cell, hbar,
      Wq, bq2, WkT, bkT, W_vw, bvw2, iw2)

    return (lp.reshape(_B), rew.reshape(_B),
            jnp.transpose(act[:, :, 0]).astype(jnp.int32))
